# Initial kernel scaffold; baseline (speedup 1.0000x reference)
import jax
import jax.numpy as jnp
from jax.experimental import pallas as pl


def kernel(x, batch_index, W1, b1, W2, b2):
    raise NotImplementedError("write your pallas kernel here")


# speedup vs baseline: 9.5702x; 9.5702x over previous
import functools

import jax
import jax.numpy as jnp
from jax.experimental import pallas as pl
from jax.experimental.pallas import tpu as pltpu

N = 32768
D = 512
A = 256
NUM_SEG = 16
BLK_R = 2048
NUM_BLK = N // BLK_R

_NEG_INF = float("-inf")


def _attn_kernel(x_ref, ids_ref, w1_ref, b1_ref, w2_ref, b2_ref, out_ref,
                 m_ref, l_ref, acc_ref):
    i = pl.program_id(0)

    @pl.when(i == 0)
    def _init():
        m_ref[...] = jnp.full((1, NUM_SEG), _NEG_INF, jnp.float32)
        l_ref[...] = jnp.zeros((1, NUM_SEG), jnp.float32)
        acc_ref[...] = jnp.zeros((NUM_SEG, D), jnp.float32)

    x = x_ref[...]
    h = jnp.dot(x, w1_ref[...], preferred_element_type=jnp.float32)
    t = jnp.tanh(h + b1_ref[...])
    s = jnp.dot(t, w2_ref[...], preferred_element_type=jnp.float32)
    s = s + b2_ref[...]

    ids_col = ids_ref[...].reshape(BLK_R, 1)
    seg_iota = jax.lax.broadcasted_iota(jnp.int32, (BLK_R, NUM_SEG), 1)
    onehot = ids_col == seg_iota

    neg = jnp.float32(_NEG_INF)
    blk_m = jnp.max(jnp.where(onehot, s, neg), axis=0, keepdims=True)
    m_old = m_ref[...]
    m_new = jnp.maximum(m_old, blk_m)
    alpha = jnp.where(m_new > neg, jnp.exp(m_old - m_new), 0.0)

    p = jnp.exp(jnp.where(onehot, s - m_new, neg))
    blk_l = jnp.sum(p, axis=0, keepdims=True)
    blk_acc = jax.lax.dot_general(
        p, x, (((0,), (0,)), ((), ())),
        preferred_element_type=jnp.float32)

    m_ref[...] = m_new
    l_ref[...] = l_ref[...] * alpha + blk_l
    acc_ref[...] = acc_ref[...] * alpha.reshape(NUM_SEG, 1) + blk_acc

    @pl.when(i == NUM_BLK - 1)
    def _fin():
        l = l_ref[...].reshape(NUM_SEG, 1)
        out_ref[...] = jnp.where(l > 0, acc_ref[...] / l, 0.0)


@functools.partial(jax.jit, static_argnames=("interpret",))
def _run(x, ids3, W1, b1r, W2, b2r, interpret=False):
    return pl.pallas_call(
        _attn_kernel,
        grid=(NUM_BLK,),
        in_specs=[
            pl.BlockSpec((BLK_R, D), lambda i: (i, 0)),
            pl.BlockSpec((1, 1, BLK_R), lambda i: (i, 0, 0)),
            pl.BlockSpec((D, A), lambda i: (0, 0)),
            pl.BlockSpec((1, A), lambda i: (0, 0)),
            pl.BlockSpec((A, 1), lambda i: (0, 0)),
            pl.BlockSpec((1, 1), lambda i: (0, 0)),
        ],
        out_specs=pl.BlockSpec((NUM_SEG, D), lambda i: (0, 0)),
        out_shape=jax.ShapeDtypeStruct((NUM_SEG, D), jnp.float32),
        scratch_shapes=[
            pltpu.VMEM((1, NUM_SEG), jnp.float32),
            pltpu.VMEM((1, NUM_SEG), jnp.float32),
            pltpu.VMEM((NUM_SEG, D), jnp.float32),
        ],
        interpret=interpret,
    )(x, ids3, W1, b1r, W2, b2r)


def kernel(x, batch_index, W1, b1, W2, b2):
    ids3 = batch_index.astype(jnp.int32).reshape(NUM_BLK, 1, BLK_R)
    b1r = b1.reshape(1, A)
    b2r = b2.reshape(1, 1)
    return _run(x, ids3, W1, b1r, W2, b2r)


# speedup vs baseline: 9.9494x; 1.0396x over previous
import functools

import jax
import jax.numpy as jnp
from jax.experimental import pallas as pl
from jax.experimental.pallas import tpu as pltpu

N = 32768
D = 512
A = 256
NUM_SEG = 16
BLK_R = 4096
NUM_BLK = N // BLK_R

_NEG_INF = float("-inf")


def _attn_kernel(x_ref, ids_ref, w1_ref, b1_ref, w2_ref, b2_ref, out_ref,
                 m_ref, l_ref, acc_ref):
    i = pl.program_id(0)

    @pl.when(i == 0)
    def _init():
        m_ref[...] = jnp.full((1, NUM_SEG), _NEG_INF, jnp.float32)
        l_ref[...] = jnp.zeros((1, NUM_SEG), jnp.float32)
        acc_ref[...] = jnp.zeros((NUM_SEG, D), jnp.float32)

    x = x_ref[...]
    xb = x.astype(jnp.bfloat16)
    h = jnp.dot(xb, w1_ref[...].astype(jnp.bfloat16),
                preferred_element_type=jnp.float32)
    t = jnp.tanh(h + b1_ref[...])
    s = jnp.dot(t.astype(jnp.bfloat16), w2_ref[...].astype(jnp.bfloat16),
                preferred_element_type=jnp.float32)
    s = s + b2_ref[...]

    ids_col = ids_ref[...].reshape(BLK_R, 1)
    seg_iota = jax.lax.broadcasted_iota(jnp.int32, (BLK_R, NUM_SEG), 1)
    onehot = ids_col == seg_iota

    neg = jnp.float32(_NEG_INF)
    blk_m = jnp.max(jnp.where(onehot, s, neg), axis=0, keepdims=True)
    m_old = m_ref[...]
    m_new = jnp.maximum(m_old, blk_m)
    alpha = jnp.where(m_new > neg, jnp.exp(m_old - m_new), 0.0)

    p = jnp.exp(jnp.where(onehot, s - m_new, neg))
    blk_l = jnp.sum(p, axis=0, keepdims=True)
    blk_acc = jax.lax.dot_general(
        p.astype(jnp.bfloat16), xb, (((0,), (0,)), ((), ())),
        preferred_element_type=jnp.float32)

    m_ref[...] = m_new
    l_ref[...] = l_ref[...] * alpha + blk_l
    acc_ref[...] = acc_ref[...] * alpha.reshape(NUM_SEG, 1) + blk_acc

    @pl.when(i == NUM_BLK - 1)
    def _fin():
        l = l_ref[...].reshape(NUM_SEG, 1)
        out_ref[...] = jnp.where(l > 0, acc_ref[...] / l, 0.0)


@functools.partial(jax.jit, static_argnames=("interpret",))
def _run(x, ids3, W1, b1r, W2, b2r, interpret=False):
    return pl.pallas_call(
        _attn_kernel,
        grid=(NUM_BLK,),
        in_specs=[
            pl.BlockSpec((BLK_R, D), lambda i: (i, 0)),
            pl.BlockSpec((1, 1, BLK_R), lambda i: (i, 0, 0)),
            pl.BlockSpec((D, A), lambda i: (0, 0)),
            pl.BlockSpec((1, A), lambda i: (0, 0)),
            pl.BlockSpec((A, 1), lambda i: (0, 0)),
            pl.BlockSpec((1, 1), lambda i: (0, 0)),
        ],
        out_specs=pl.BlockSpec((NUM_SEG, D), lambda i: (0, 0)),
        out_shape=jax.ShapeDtypeStruct((NUM_SEG, D), jnp.float32),
        scratch_shapes=[
            pltpu.VMEM((1, NUM_SEG), jnp.float32),
            pltpu.VMEM((1, NUM_SEG), jnp.float32),
            pltpu.VMEM((NUM_SEG, D), jnp.float32),
        ],
        interpret=interpret,
    )(x, ids3, W1, b1r, W2, b2r)


def kernel(x, batch_index, W1, b1, W2, b2):
    ids3 = batch_index.astype(jnp.int32).reshape(NUM_BLK, 1, BLK_R)
    b1r = b1.reshape(1, A)
    b2r = b2.reshape(1, 1)
    return _run(x, ids3, W1, b1r, W2, b2r)


# speedup vs baseline: 10.1514x; 1.0203x over previous
import functools

import jax
import jax.numpy as jnp
from jax.experimental import pallas as pl
from jax.experimental.pallas import tpu as pltpu

N = 32768
D = 512
A = 256
NUM_SEG = 16
BLK_R = 4096
NUM_BLK = N // BLK_R


def _attn_kernel(x_ref, ids_ref, w1_ref, b1_ref, w2_ref, c_ref, out_ref,
                 l_ref, acc_ref):
    i = pl.program_id(0)

    @pl.when(i == 0)
    def _init():
        l_ref[...] = jnp.zeros((1, NUM_SEG), jnp.float32)
        acc_ref[...] = jnp.zeros((NUM_SEG, D), jnp.float32)

    x = x_ref[...]
    xb = x.astype(jnp.bfloat16)
    h = jnp.dot(xb, w1_ref[...], preferred_element_type=jnp.float32)
    t = jnp.tanh(h + b1_ref[...])
    s16 = jnp.dot(t.astype(jnp.bfloat16), w2_ref[...],
                  preferred_element_type=jnp.float32)
    e16 = jnp.exp(s16 + c_ref[...])

    ids_col = ids_ref[...].reshape(BLK_R, 1)
    seg_iota = jax.lax.broadcasted_iota(jnp.int32, (BLK_R, NUM_SEG), 1)
    p = jnp.where(ids_col == seg_iota, e16, 0.0)

    blk_l = jnp.sum(p, axis=0, keepdims=True)
    blk_acc = jax.lax.dot_general(
        p.astype(jnp.bfloat16), xb, (((0,), (0,)), ((), ())),
        preferred_element_type=jnp.float32)

    l_ref[...] = l_ref[...] + blk_l
    acc_ref[...] = acc_ref[...] + blk_acc

    @pl.when(i == NUM_BLK - 1)
    def _fin():
        l = l_ref[...].reshape(NUM_SEG, 1)
        out_ref[...] = jnp.where(l > 0, acc_ref[...] / l, 0.0)


@functools.partial(jax.jit, static_argnames=("interpret",))
def _run(x, ids3, W1b, b1r, W2r, c, interpret=False):
    return pl.pallas_call(
        _attn_kernel,
        grid=(NUM_BLK,),
        in_specs=[
            pl.BlockSpec((BLK_R, D), lambda i: (i, 0)),
            pl.BlockSpec((1, 1, BLK_R), lambda i: (i, 0, 0)),
            pl.BlockSpec((D, A), lambda i: (0, 0)),
            pl.BlockSpec((1, A), lambda i: (0, 0)),
            pl.BlockSpec((A, NUM_SEG), lambda i: (0, 0)),
            pl.BlockSpec((1, NUM_SEG), lambda i: (0, 0)),
        ],
        out_specs=pl.BlockSpec((NUM_SEG, D), lambda i: (0, 0)),
        out_shape=jax.ShapeDtypeStruct((NUM_SEG, D), jnp.float32),
        scratch_shapes=[
            pltpu.VMEM((1, NUM_SEG), jnp.float32),
            pltpu.VMEM((NUM_SEG, D), jnp.float32),
        ],
        interpret=interpret,
    )(x, ids3, W1b, b1r, W2r, c)


def kernel(x, batch_index, W1, b1, W2, b2):
    ids3 = batch_index.astype(jnp.int32).reshape(NUM_BLK, 1, BLK_R)
    bound = jnp.sum(jnp.abs(W2))
    c = jnp.broadcast_to((b2 - bound).reshape(1, 1), (1, NUM_SEG))
    W2r = jnp.broadcast_to(W2.astype(jnp.bfloat16), (A, NUM_SEG))
    return _run(x, ids3, W1.astype(jnp.bfloat16), b1.reshape(1, A), W2r, c)
